# baseline (speedup 1.0000x reference)
import functools

import jax
import jax.numpy as jnp
from jax import lax
from jax.experimental import pallas as pl
from jax.experimental.pallas import tpu as pltpu
from jax.experimental.pallas import tpu_sc as plsc

N = 4096
D = 1024
DFF = 2048
E = 16
K = 2
PAIRS = N * K
BLK = 512
NB = PAIRS // BLK + E
P = NB * BLK
DFFC = 512
NJ = DFF // DFFC

NC = 2
NS = 16
NW = NC * NS
PAIRS_PER_W = PAIRS // NW
TOK_PER_W = N // NW
CH = 32
NCH = PAIRS_PER_W // CH


def _gate_body(x_ref, wg_ref, bg_ref, topk_ref, pos_ref, be_ref):
    x = x_ref[...]
    logits = lax.dot_general(
        x, wg_ref[...], (((1,), (0,)), ((), ())),
    ) + bg_ref[...]
    iota = lax.broadcasted_iota(jnp.int32, (N, E), 1)

    m1 = jnp.max(logits, axis=1, keepdims=True)
    idx1 = jnp.min(jnp.where(logits == m1, iota, E), axis=1, keepdims=True)
    l2 = jnp.where(iota == idx1, -jnp.inf, logits)
    m2 = jnp.max(l2, axis=1, keepdims=True)
    idx2 = jnp.min(jnp.where(l2 == m2, iota, E), axis=1, keepdims=True)
    topk_ref[...] = jnp.concatenate([idx1, idx2], axis=1)

    oh = (iota == idx1).astype(jnp.float32) + (iota == idx2).astype(jnp.float32)
    s = oh
    k = 1
    while k < N:
        s = s + jnp.concatenate([jnp.zeros((k, E), jnp.float32), s[:-k]], axis=0)
        k *= 2
    c_excl = s - oh
    counts = s[N - 1:N, :]
    aligned = jnp.ceil(counts / BLK) * BLK
    te = lax.broadcasted_iota(jnp.int32, (E, E), 0)
    ce = lax.broadcasted_iota(jnp.int32, (E, E), 1)
    tri = (te < ce).astype(jnp.float32)
    base = lax.dot_general(aligned, tri, (((1,), (0,)), ((), ())))
    ends = base + aligned

    posf = base + c_excl
    pos1 = jnp.sum(jnp.where(iota == idx1, posf, 0.0), axis=1, keepdims=True)
    pos2 = jnp.sum(jnp.where(iota == idx2, posf, 0.0), axis=1, keepdims=True)
    pos_ref[...] = (
        jnp.concatenate([pos1, pos2], axis=1) + 0.5
    ).astype(jnp.int32)

    bidx = (lax.broadcasted_iota(jnp.int32, (8, 128), 0) * 128
            + lax.broadcasted_iota(jnp.int32, (8, 128), 1))
    bstart = (bidx * BLK).astype(jnp.float32)
    beval = jnp.sum(
        (bstart[:, :, None] >= ends.reshape(1, 1, E)).astype(jnp.int32), axis=2)
    valid = bstart < jnp.max(ends)
    be_ref[...] = jnp.where(valid, beval, -1)


def _gate_call(moe_inp, Wg, bg):
    return pl.pallas_call(
        _gate_body,
        out_shape=(
            jax.ShapeDtypeStruct((N, K), jnp.int32),
            jax.ShapeDtypeStruct((N, K), jnp.int32),
            jax.ShapeDtypeStruct((8, 128), jnp.int32),
        ),
    )(moe_inp, Wg, bg.reshape(1, E))


@functools.lru_cache(maxsize=None)
def _sc_kernels():
    mesh = plsc.VectorSubcoreMesh(core_axis_name="c", subcore_axis_name="s")

    @functools.partial(
        pl.kernel,
        out_type=jax.ShapeDtypeStruct((P, D // 2), jnp.int32),
        mesh=mesh,
        scratch_types=[
            pltpu.VMEM((NCH, CH), jnp.int32),
            pltpu.VMEM((NCH, CH), jnp.int32),
            pltpu.VMEM((CH, D // 2), jnp.int32),
            pltpu.VMEM((CH, D // 2), jnp.int32),
            pltpu.SemaphoreType.DMA,
            pltpu.SemaphoreType.DMA,
            pltpu.SemaphoreType.DMA,
            pltpu.SemaphoreType.DMA,
        ],
    )
    def _dispatch(x_hbm, tok_hbm, pos_hbm, xs_hbm, tok_v, pos_v,
                  rows0, rows1, sg0, sg1, ss0, ss1):
        wid = lax.axis_index("s") * NC + lax.axis_index("c")
        pltpu.sync_copy(tok_hbm.at[pl.ds(wid * NCH, NCH)], tok_v)
        pltpu.sync_copy(pos_hbm.at[pl.ds(wid * NCH, NCH)], pos_v)
        rows = (rows0, rows1)
        sg = (sg0, sg1)
        ss = (ss0, ss1)
        hg = [None, None]
        hs = [None, None]
        hg[0] = pltpu.async_copy(x_hbm.at[tok_v.at[0]], rows[0], sg[0])
        for c in range(NCH):
            buf = c % 2
            nbuf = (c + 1) % 2
            if c + 1 < NCH:
                if hs[nbuf] is not None:
                    hs[nbuf].wait()
                hg[nbuf] = pltpu.async_copy(
                    x_hbm.at[tok_v.at[c + 1]], rows[nbuf], sg[nbuf])
            hg[buf].wait()
            hs[buf] = pltpu.async_copy(
                rows[buf], xs_hbm.at[pos_v.at[c]], ss[buf])
        hs[0].wait()
        hs[1].wait()

    @functools.partial(
        pl.kernel,
        out_type=jax.ShapeDtypeStruct((N, D), jnp.float32),
        mesh=mesh,
        scratch_types=[
            pltpu.VMEM((NCH, CH), jnp.int32),
            pltpu.VMEM((CH, D), jnp.float32),
            pltpu.VMEM((CH, D), jnp.float32),
            pltpu.VMEM((CH // K, D), jnp.float32),
            pltpu.VMEM((CH // K, D), jnp.float32),
            pltpu.SemaphoreType.DMA,
            pltpu.SemaphoreType.DMA,
            pltpu.SemaphoreType.DMA,
            pltpu.SemaphoreType.DMA,
        ],
    )
    def _combine(os_hbm, pos_hbm, y_hbm, pos_v, rows0, rows1,
                 out0, out1, sg0, sg1, so0, so1):
        wid = lax.axis_index("s") * NC + lax.axis_index("c")
        pltpu.sync_copy(pos_hbm.at[pl.ds(wid * NCH, NCH)], pos_v)
        rows = (rows0, rows1)
        outs = (out0, out1)
        sg = (sg0, sg1)
        so = (so0, so1)
        hg = [None, None]
        ho = [None, None]
        hg[0] = pltpu.async_copy(os_hbm.at[pos_v.at[0]], rows[0], sg[0])
        for c in range(NCH):
            buf = c % 2
            nbuf = (c + 1) % 2
            if c + 1 < NCH:
                hg[nbuf] = pltpu.async_copy(
                    os_hbm.at[pos_v.at[c + 1]], rows[nbuf], sg[nbuf])
            hg[buf].wait()
            if ho[buf] is not None:
                ho[buf].wait()
            rv = rows[buf]
            ov = outs[buf]

            def _body(t, carry):
                for v in range(D // 16):
                    sl = pl.ds(v * 16, 16)
                    ov[t, sl] = rv[2 * t, sl] + rv[2 * t + 1, sl]
                return carry

            lax.fori_loop(0, CH // K, _body, 0)
            ho[buf] = pltpu.async_copy(
                ov, y_hbm.at[pl.ds((wid * NCH + c) * (CH // K), CH // K)],
                so[buf])
        ho[0].wait()
        ho[1].wait()

    return _dispatch, _combine


def _ffn_body(be_ref, xs_ref, w1_ref, b1_ref, w2_ref, b2_ref, os_ref):
    b = pl.program_id(0)
    e = be_ref[b]

    @pl.when(e >= 0)
    def _run():
        x = xs_ref[...].astype(jnp.float32)
        h = jnp.maximum(
            lax.dot_general(x, w1_ref[0], (((1,), (0,)), ((), ())),
                            preferred_element_type=jnp.float32)
            + b1_ref[0], 0.0)
        os_ref[...] = lax.dot_general(
            h, w2_ref[0], (((1,), (0,)), ((), ())),
            preferred_element_type=jnp.float32) + b2_ref[0]


def _ffn_grid_spec():
    def _e(be, b):
        e = be[b]
        return jnp.where(e < 0, E - 1, e)

    return pltpu.PrefetchScalarGridSpec(
        num_scalar_prefetch=1,
        grid=(NB,),
        in_specs=[
            pl.BlockSpec((BLK, D), lambda b, be: (b, 0)),
            pl.BlockSpec((1, D, DFF), lambda b, be: (_e(be, b), 0, 0)),
            pl.BlockSpec((1, 1, DFF), lambda b, be: (_e(be, b), 0, 0)),
            pl.BlockSpec((1, DFF, D), lambda b, be: (_e(be, b), 0, 0)),
            pl.BlockSpec((1, 1, D), lambda b, be: (_e(be, b), 0, 0)),
        ],
        out_specs=pl.BlockSpec((BLK, D), lambda b, be: (b, 0)),
    )


def _ffn_call(be_vec, xs, W1, b1, W2, b2):
    return pl.pallas_call(
        _ffn_body,
        grid_spec=_ffn_grid_spec(),
        out_shape=jax.ShapeDtypeStruct((P, D), jnp.float32),
        compiler_params=pltpu.CompilerParams(
            dimension_semantics=("arbitrary",)),
    )(be_vec, xs, W1, b1.reshape(E, 1, DFF), W2, b2.reshape(E, 1, D))


def _to_i32(a16):
    n, d = a16.shape
    return lax.bitcast_convert_type(a16.reshape(n, d // 2, 2), jnp.int32)


def _to_bf16(a32):
    n, d2 = a32.shape
    return lax.bitcast_convert_type(a32, jnp.bfloat16).reshape(n, d2 * 2)


def kernel(moe_inp, Wg, bg, W1, b1, W2, b2):
    dispatch, combine = _sc_kernels()
    topk_idx, pos, be2d = _gate_call(moe_inp, Wg, bg)
    pos2d = pos.reshape(PAIRS // CH, CH)
    be_vec = be2d.reshape(-1)[:NB]
    tok2d = (jnp.arange(PAIRS, dtype=jnp.int32) // K).reshape(PAIRS // CH, CH)
    xs32 = dispatch(_to_i32(moe_inp.astype(jnp.bfloat16)), tok2d, pos2d)
    os_ = _ffn_call(be_vec, _to_bf16(xs32), W1, b1, W2, b2)
    y = combine(os_, pos2d)
    return y, topk_idx


# speedup vs baseline: 2.1156x; 2.1156x over previous
import functools

import jax
import jax.numpy as jnp
from jax import lax
from jax.experimental import pallas as pl
from jax.experimental.pallas import tpu as pltpu
from jax.experimental.pallas import tpu_sc as plsc

N = 4096
D = 1024
DFF = 2048
E = 16
K = 2
PAIRS = N * K
BLK = 512
NB = PAIRS // BLK + E
P = NB * BLK
DFFC = 512
NJ = DFF // DFFC

NC = 2
NS = 16
NW = NC * NS
PAIRS_PER_W = PAIRS // NW
TOK_PER_W = N // NW
CH = 32
NCH = PAIRS_PER_W // CH


def _gate_body(x_ref, wg_ref, bg_ref, topk_ref, pos_ref, be_ref):
    x = x_ref[...]
    logits = lax.dot_general(
        x, wg_ref[...], (((1,), (0,)), ((), ())),
    ) + bg_ref[...]
    iota = lax.broadcasted_iota(jnp.int32, (N, E), 1)

    m1 = jnp.max(logits, axis=1, keepdims=True)
    idx1 = jnp.min(jnp.where(logits == m1, iota, E), axis=1, keepdims=True)
    l2 = jnp.where(iota == idx1, -jnp.inf, logits)
    m2 = jnp.max(l2, axis=1, keepdims=True)
    idx2 = jnp.min(jnp.where(l2 == m2, iota, E), axis=1, keepdims=True)
    topk_ref[...] = jnp.concatenate([idx1, idx2], axis=1)

    oh = (iota == idx1).astype(jnp.float32) + (iota == idx2).astype(jnp.float32)
    s = oh
    k = 1
    while k < N:
        s = s + jnp.concatenate([jnp.zeros((k, E), jnp.float32), s[:-k]], axis=0)
        k *= 2
    c_excl = s - oh
    counts = s[N - 1:N, :]
    aligned = jnp.ceil(counts / BLK) * BLK
    te = lax.broadcasted_iota(jnp.int32, (E, E), 0)
    ce = lax.broadcasted_iota(jnp.int32, (E, E), 1)
    tri = (te < ce).astype(jnp.float32)
    base = lax.dot_general(aligned, tri, (((1,), (0,)), ((), ())))
    ends = base + aligned

    posf = base + c_excl
    pos1 = jnp.sum(jnp.where(iota == idx1, posf, 0.0), axis=1, keepdims=True)
    pos2 = jnp.sum(jnp.where(iota == idx2, posf, 0.0), axis=1, keepdims=True)
    pos_ref[...] = (
        jnp.concatenate([pos1, pos2], axis=1) + 0.5
    ).astype(jnp.int32)

    bidx = (lax.broadcasted_iota(jnp.int32, (8, 128), 0) * 128
            + lax.broadcasted_iota(jnp.int32, (8, 128), 1))
    bstart = (bidx * BLK).astype(jnp.float32)
    beval = jnp.sum(
        (bstart[:, :, None] >= ends.reshape(1, 1, E)).astype(jnp.int32), axis=2)
    valid = bstart < jnp.max(ends)
    be_ref[...] = jnp.where(valid, beval, -1)


def _gate_call(moe_inp, Wg, bg):
    return pl.pallas_call(
        _gate_body,
        out_shape=(
            jax.ShapeDtypeStruct((N, K), jnp.int32),
            jax.ShapeDtypeStruct((N, K), jnp.int32),
            jax.ShapeDtypeStruct((8, 128), jnp.int32),
        ),
    )(moe_inp, Wg, bg.reshape(1, E))


@functools.lru_cache(maxsize=None)
def _sc_kernels():
    mesh = plsc.VectorSubcoreMesh(core_axis_name="c", subcore_axis_name="s")

    @functools.partial(
        pl.kernel,
        out_type=jax.ShapeDtypeStruct((P, D), jnp.float32),
        mesh=mesh,
        scratch_types=[
            pltpu.VMEM((NCH, CH), jnp.int32),
            pltpu.VMEM((NCH, CH), jnp.int32),
            pltpu.VMEM((CH, D), jnp.float32),
            pltpu.VMEM((CH, D), jnp.float32),
            pltpu.SemaphoreType.DMA,
            pltpu.SemaphoreType.DMA,
            pltpu.SemaphoreType.DMA,
            pltpu.SemaphoreType.DMA,
        ],
    )
    def _dispatch(x_hbm, tok_hbm, pos_hbm, xs_hbm, tok_v, pos_v,
                  rows0, rows1, sg0, sg1, ss0, ss1):
        wid = lax.axis_index("s") * NC + lax.axis_index("c")
        pltpu.sync_copy(tok_hbm.at[pl.ds(wid * NCH, NCH)], tok_v)
        pltpu.sync_copy(pos_hbm.at[pl.ds(wid * NCH, NCH)], pos_v)
        rows = (rows0, rows1)
        sg = (sg0, sg1)
        ss = (ss0, ss1)
        hg = [None, None]
        hs = [None, None]
        hg[0] = pltpu.async_copy(x_hbm.at[tok_v.at[0]], rows[0], sg[0])
        for c in range(NCH):
            buf = c % 2
            nbuf = (c + 1) % 2
            if c + 1 < NCH:
                if hs[nbuf] is not None:
                    hs[nbuf].wait()
                hg[nbuf] = pltpu.async_copy(
                    x_hbm.at[tok_v.at[c + 1]], rows[nbuf], sg[nbuf])
            hg[buf].wait()
            hs[buf] = pltpu.async_copy(
                rows[buf], xs_hbm.at[pos_v.at[c]], ss[buf])
        hs[0].wait()
        hs[1].wait()

    @functools.partial(
        pl.kernel,
        out_type=jax.ShapeDtypeStruct((N, D), jnp.float32),
        mesh=mesh,
        scratch_types=[
            pltpu.VMEM((NCH, CH), jnp.int32),
            pltpu.VMEM((CH, D), jnp.float32),
            pltpu.VMEM((CH, D), jnp.float32),
            pltpu.VMEM((CH // K, D), jnp.float32),
            pltpu.VMEM((CH // K, D), jnp.float32),
            pltpu.SemaphoreType.DMA,
            pltpu.SemaphoreType.DMA,
            pltpu.SemaphoreType.DMA,
            pltpu.SemaphoreType.DMA,
        ],
    )
    def _combine(os_hbm, pos_hbm, y_hbm, pos_v, rows0, rows1,
                 out0, out1, sg0, sg1, so0, so1):
        wid = lax.axis_index("s") * NC + lax.axis_index("c")
        pltpu.sync_copy(pos_hbm.at[pl.ds(wid * NCH, NCH)], pos_v)
        rows = (rows0, rows1)
        outs = (out0, out1)
        sg = (sg0, sg1)
        so = (so0, so1)
        hg = [None, None]
        ho = [None, None]
        hg[0] = pltpu.async_copy(os_hbm.at[pos_v.at[0]], rows[0], sg[0])
        for c in range(NCH):
            buf = c % 2
            nbuf = (c + 1) % 2
            if c + 1 < NCH:
                hg[nbuf] = pltpu.async_copy(
                    os_hbm.at[pos_v.at[c + 1]], rows[nbuf], sg[nbuf])
            hg[buf].wait()
            if ho[buf] is not None:
                ho[buf].wait()
            rv = rows[buf]
            ov = outs[buf]

            def _body(t, carry):
                for v in range(D // 16):
                    sl = pl.ds(v * 16, 16)
                    ov[t, sl] = rv[2 * t, sl] + rv[2 * t + 1, sl]
                return carry

            lax.fori_loop(0, CH // K, _body, 0)
            ho[buf] = pltpu.async_copy(
                ov, y_hbm.at[pl.ds((wid * NCH + c) * (CH // K), CH // K)],
                so[buf])
        ho[0].wait()
        ho[1].wait()

    return _dispatch, _combine


def _ffn_body(be_ref, xs_ref, w1_ref, b1_ref, w2_ref, b2_ref, os_ref):
    b = pl.program_id(0)
    e = be_ref[0, b]

    @pl.when(e >= 0)
    def _run():
        x = xs_ref[...]
        h = jnp.maximum(
            lax.dot_general(x, w1_ref[0], (((1,), (0,)), ((), ())),
                            preferred_element_type=jnp.float32)
            + b1_ref[0], 0.0)
        os_ref[...] = lax.dot_general(
            h, w2_ref[0], (((1,), (0,)), ((), ())),
            preferred_element_type=jnp.float32) + b2_ref[0]


def _ffn_grid_spec():
    def _e(be, b):
        e = be[0, b]
        return jnp.where(e < 0, E - 1, e)

    return pltpu.PrefetchScalarGridSpec(
        num_scalar_prefetch=1,
        grid=(NB,),
        in_specs=[
            pl.BlockSpec((BLK, D), lambda b, be: (b, 0)),
            pl.BlockSpec((1, D, DFF), lambda b, be: (_e(be, b), 0, 0)),
            pl.BlockSpec((1, 1, DFF), lambda b, be: (_e(be, b), 0, 0)),
            pl.BlockSpec((1, DFF, D), lambda b, be: (_e(be, b), 0, 0)),
            pl.BlockSpec((1, 1, D), lambda b, be: (_e(be, b), 0, 0)),
        ],
        out_specs=pl.BlockSpec((BLK, D), lambda b, be: (b, 0)),
    )


def _ffn_call(be_vec, xs, W1, b1, W2, b2):
    return pl.pallas_call(
        _ffn_body,
        grid_spec=_ffn_grid_spec(),
        out_shape=jax.ShapeDtypeStruct((P, D), jnp.float32),
        compiler_params=pltpu.CompilerParams(
            dimension_semantics=("arbitrary",)),
    )(be_vec, xs, W1, b1.reshape(E, 1, DFF), W2, b2.reshape(E, 1, D))


def kernel(moe_inp, Wg, bg, W1, b1, W2, b2):
    dispatch, combine = _sc_kernels()
    topk_idx, pos, be2d = _gate_call(moe_inp, Wg, bg)
    pos2d = pos.reshape(PAIRS // CH, CH)
    tok2d = (jnp.arange(PAIRS, dtype=jnp.int32) // K).reshape(PAIRS // CH, CH)
    xs = dispatch(moe_inp, tok2d, pos2d)
    os_ = _ffn_call(be2d, xs, W1, b1, W2, b2)
    y = combine(os_, pos2d)
    return y, topk_idx
